# Initial kernel scaffold; baseline (speedup 1.0000x reference)
import jax
import jax.numpy as jnp
from jax.experimental import pallas as pl


def kernel(indices, lengths, embeddings):
    raise NotImplementedError("write your pallas kernel here")


# speedup vs baseline: 1.8174x; 1.8174x over previous
import functools

import jax
import jax.numpy as jnp
from jax import lax
from jax.experimental import pallas as pl
from jax.experimental.pallas import tpu as pltpu
from jax.experimental.pallas import tpu_sc as plsc

_GATHER_WINDOW = 128


def _sc_gather(embeddings, idx_flat):
    n = idx_flat.shape[0]
    d = embeddings.shape[1]
    w = _GATHER_WINDOW
    assert n % w == 0
    mesh = plsc.VectorSubcoreMesh(core_axis_name="core", subcore_axis_name="subcore")
    idx2 = idx_flat.reshape(1, n)

    @functools.partial(
        pl.kernel,
        out_type=jax.ShapeDtypeStruct((n, d), embeddings.dtype),
        mesh=mesh,
        compiler_params=pltpu.CompilerParams(use_tc_tiling_on_sc=False),
    )
    def k(x_hbm, i_hbm, o_hbm):
        def body(i_vmem, o_vmem):
            pltpu.sync_copy(x_hbm.at[i_vmem.at[0]], o_vmem)

        pltpu.emit_pipeline(
            body,
            grid=(n // w,),
            in_specs=[pl.BlockSpec((1, w), index_map=lambda i: (0, i))],
            out_specs=[pl.BlockSpec((w, d), index_map=lambda i: (i, 0))],
            core_axis_name=("core", "subcore"),
            dimension_semantics=(pltpu.PARALLEL,),
        )(i_hbm, o_hbm)

    return k(embeddings, idx2)


def _tc_mask(lengths, batch, seq):
    def mk(len_ref, out_ref):
        pos = lax.broadcasted_iota(jnp.int32, out_ref.shape, 1)
        out_ref[...] = (pos < len_ref[...]).astype(jnp.int8)

    rows = 128
    return pl.pallas_call(
        mk,
        grid=(batch // rows,),
        in_specs=[pl.BlockSpec((rows, 1), lambda i: (i, 0))],
        out_specs=pl.BlockSpec((rows, seq), lambda i: (i, 0)),
        out_shape=jax.ShapeDtypeStruct((batch, seq), jnp.int8),
    )(lengths.reshape(batch, 1))


def kernel(indices, lengths, embeddings):
    batch, seq = indices.shape
    d = embeddings.shape[1]
    emb_flat = _sc_gather(embeddings, indices.reshape(-1))
    emb_words = emb_flat.reshape(batch, seq, d)
    att_words = _tc_mask(lengths, batch, seq).astype(jnp.bool_)
    return (emb_words, att_words)
